# baseline (speedup 1.0000x reference)
import functools
import math

import jax
import jax.numpy as jnp
from jax import lax
from jax.experimental import pallas as pl
from jax.experimental.pallas import tpu as pltpu
from jax.experimental.pallas import tpu_sc as plsc

_NC = 2
_NS = 16
_CH = 128
_GRX = 4
_GRT = 8


def _acc_plan(n):
    nacc = ((n + 1 + 7) // 8) * 8
    zrpt = (nacc // _NS) & ~7
    zrpt_last = nacc - zrpt * (_NS - 1)
    rpt = (n // _NS) & ~7
    rpt_last = n - rpt * (_NS - 1)
    return nacc, zrpt, zrpt_last, rpt, rpt_last


def _zero_acc(z_hbm, stage_v, acc, s, zrpt, zrpt_last):
    pltpu.sync_copy(z_hbm, stage_v)

    def zero_rows(zr0, nrows):
        zfull, zrem = divmod(nrows, _CH)
        for k in range(zfull):
            pltpu.sync_copy(stage_v, acc.at[pl.ds(zr0 + _CH * k, _CH)])
        if zrem:
            pltpu.sync_copy(stage_v.at[pl.ds(0, zrem)],
                            acc.at[pl.ds(zr0 + _CH * zfull, zrem)])

    @pl.when(s < _NS - 1)
    def _():
        zero_rows(s * zrpt, zrpt)

    @pl.when(s == _NS - 1)
    def _():
        zero_rows((_NS - 1) * zrpt, zrpt_last)


@functools.partial(jax.jit, static_argnames=("n", "epad", "dfh"))
def _sc_segsum_x(xr, sdb, n, epad, dfh):
    nacc, zrpt, zrpt_last, rpt, rpt_last = _acc_plan(n)
    gpt = epad // (_CH * _GRX * _NS)
    mesh = plsc.VectorSubcoreMesh(core_axis_name="c", subcore_axis_name="s")

    @functools.partial(
        pl.kernel,
        out_type=jax.ShapeDtypeStruct((_NC, n, dfh), jnp.float32),
        mesh=mesh,
        scratch_types=[
            pltpu.VMEM((2 * _GRX, _CH), jnp.int32),
            pltpu.VMEM((_CH, dfh), jnp.float32),
            pltpu.VMEM((_CH, dfh), jnp.float32),
            pltpu.VMEM((_CH, dfh), jnp.float32),
            pltpu.SemaphoreType.DMA,
            pltpu.SemaphoreType.DMA,
            pltpu.SemaphoreType.DMA,
            pltpu.SemaphoreType.DMA,
            pltpu.SemaphoreType.DMA,
            pltpu.SemaphoreType.DMA,
            pltpu.VMEM_SHARED((nacc, dfh), jnp.float32),
        ],
    )
    def sc_kernel(xr_hbm, sdb_hbm, z_hbm, s_out,
                  idx_v, b0, b1, b2, gs0, gs1, gs2, ss0, ss1, ss2,
                  acc):
        c = lax.axis_index("c")
        s = lax.axis_index("s")
        _zero_acc(z_hbm, b0, acc, s, zrpt, zrpt_last)
        plsc.subcore_barrier()

        bufs = (b0, b1, b2)
        gsems = (gs0, gs1, gs2)
        ssems = (ss0, ss1, ss2)
        nb = 3

        def gather(j, b):
            return pltpu.async_copy(xr_hbm.at[idx_v.at[j]], bufs[b],
                                    gsems[b])

        def scatter(j, b):
            return pltpu.async_copy(bufs[b], acc.at[idx_v.at[_GRX + j]],
                                    ssems[b], add=True)

        def group(g, carry):
            pltpu.sync_copy(sdb_hbm.at[c, g], idx_v)
            gd = [gather(0, 0), gather(1, 1), None]
            sd = [None] * nb
            for k in range(_GRX):
                b = k % nb
                gd[b].wait()
                sd[b] = scatter(k, b)
                nxt = k + 2
                if nxt < _GRX:
                    bn = nxt % nb
                    if k >= 1:
                        sd[bn].wait()
                    gd[bn] = gather(nxt, bn)
            for k in range(_GRX - nb, _GRX):
                sd[k % nb].wait()
            return carry

        lax.fori_loop(s * gpt, (s + 1) * gpt, group, 0)
        plsc.subcore_barrier()

        def write_rows(row0, nrows):
            pltpu.sync_copy(acc.at[pl.ds(row0, nrows)],
                            s_out.at[c, pl.ds(row0, nrows)])

        @pl.when(s < _NS - 1)
        def _():
            write_rows(s * rpt, rpt)

        @pl.when(s == _NS - 1)
        def _():
            write_rows((_NS - 1) * rpt, rpt_last)

    zeros = jnp.zeros((_CH, dfh), jnp.float32)
    return sc_kernel(xr, sdb, zeros)


@functools.partial(jax.jit, static_argnames=("n", "epad", "dfh"))
def _sc_segsum_ea(ea128, dstb, n, epad, dfh):
    nacc, zrpt, zrpt_last, rpt, rpt_last = _acc_plan(n)
    gpw = epad // (_CH * _GRT * _NS * _NC)
    mesh = plsc.VectorSubcoreMesh(core_axis_name="c", subcore_axis_name="s")

    @functools.partial(
        pl.kernel,
        out_type=jax.ShapeDtypeStruct((_NC, n, dfh), jnp.float32),
        mesh=mesh,
        scratch_types=[
            pltpu.VMEM((_GRT, _CH), jnp.int32),
            pltpu.VMEM((_CH, dfh), jnp.float32),
            pltpu.VMEM((_CH, dfh), jnp.float32),
            pltpu.SemaphoreType.DMA,
            pltpu.SemaphoreType.DMA,
            pltpu.SemaphoreType.DMA,
            pltpu.SemaphoreType.DMA,
            pltpu.VMEM_SHARED((nacc, dfh), jnp.float32),
        ],
    )
    def sc_kernel(ea_hbm, dstb_hbm, z_hbm, t_out,
                  dst_v, rows_a, rows_b, gs0, gs1, ss0, ss1, acc):
        c = lax.axis_index("c")
        s = lax.axis_index("s")
        _zero_acc(z_hbm, rows_a, acc, s, zrpt, zrpt_last)
        plsc.subcore_barrier()

        bufs = (rows_a, rows_b)
        gsems = (gs0, gs1)
        ssems = (ss0, ss1)

        def group(g, carry):
            pltpu.sync_copy(dstb_hbm.at[g], dst_v)
            e0 = g * _GRT * _CH
            gd = [None, None]
            sd = [None, None]
            gd[0] = pltpu.async_copy(ea_hbm.at[pl.ds(e0, _CH)], bufs[0],
                                     gsems[0])
            for j in range(_GRT):
                p = j % 2
                if j + 1 < _GRT:
                    if j >= 1:
                        sd[1 - p].wait()
                    gd[1 - p] = pltpu.async_copy(
                        ea_hbm.at[pl.ds(e0 + (j + 1) * _CH, _CH)],
                        bufs[1 - p], gsems[1 - p])
                gd[p].wait()
                sd[p] = pltpu.async_copy(
                    bufs[p], acc.at[dst_v.at[j]], ssems[p], add=True)
            sd[0].wait()
            sd[1].wait()
            return carry

        w = c * _NS + s
        lax.fori_loop(w * gpw, (w + 1) * gpw, group, 0)
        plsc.subcore_barrier()

        def write_rows(row0, nrows):
            pltpu.sync_copy(acc.at[pl.ds(row0, nrows)],
                            t_out.at[c, pl.ds(row0, nrows)])

        @pl.when(s < _NS - 1)
        def _():
            write_rows(s * rpt, rpt)

        @pl.when(s == _NS - 1)
        def _():
            write_rows((_NS - 1) * rpt, rpt_last)

    zeros = jnp.zeros((_CH, dfh), jnp.float32)
    return sc_kernel(ea128, dstb, zeros)


def _mm_body(s_ref, t_ref, wx0_ref, wx1_ref, we_ref, o_ref):
    acc = jnp.dot(s_ref[0], wx0_ref[...], preferred_element_type=jnp.float32)
    acc += jnp.dot(s_ref[1], wx1_ref[...], preferred_element_type=jnp.float32)
    acc += jnp.dot(t_ref[0] + t_ref[1], we_ref[...],
                   preferred_element_type=jnp.float32)
    o_ref[...] = acc


def kernel(x, edge_index, edge_attr, W):
    n, df = x.shape
    e = edge_index.shape[1]
    de = edge_attr.shape[1]
    do = W.shape[1]
    dfh = df // 2
    assert df == 2 * dfh and n % _NS == 0 and de <= dfh

    grp_x = _NS * _GRX * _CH
    grp_t = _NC * _NS * _GRT * _CH
    grp = grp_x * grp_t // math.gcd(grp_x, grp_t)
    epad = ((e + grp - 1) // grp) * grp
    pad = epad - e
    src = edge_index[0]
    dst = edge_index[1]
    if pad:
        src = jnp.concatenate([src, jnp.zeros((pad,), jnp.int32)])
        dst = jnp.concatenate([dst, jnp.full((pad,), n, jnp.int32)])
    ngrp_x = epad // (_GRX * _CH)
    srcb = (jnp.stack([2 * src, 2 * src + 1])
            .reshape(2, ngrp_x, _GRX, _CH))
    dstb_x = jnp.broadcast_to(dst.reshape(1, ngrp_x, _GRX, _CH),
                              (2, ngrp_x, _GRX, _CH))
    sdb = jnp.concatenate([srcb, dstb_x], axis=2)
    dstb_t = dst.reshape(epad // (_GRT * _CH), _GRT, _CH)

    xr = x.reshape(2 * n, dfh)
    ea128 = jnp.zeros((epad, dfh), jnp.float32).at[:e, :de].set(edge_attr)
    we128 = jnp.zeros((dfh, do), jnp.float32).at[:de].set(W[df:])

    s_acc = _sc_segsum_x(xr, sdb, n=n, epad=epad, dfh=dfh)
    t_acc = _sc_segsum_ea(ea128, dstb_t, n=n, epad=epad, dfh=dfh)

    mb = 1000
    out = pl.pallas_call(
        _mm_body,
        grid=(n // mb,),
        in_specs=[
            pl.BlockSpec((_NC, mb, dfh), lambda i: (0, i, 0)),
            pl.BlockSpec((_NC, mb, dfh), lambda i: (0, i, 0)),
            pl.BlockSpec((dfh, do), lambda i: (0, 0)),
            pl.BlockSpec((dfh, do), lambda i: (0, 0)),
            pl.BlockSpec((dfh, do), lambda i: (0, 0)),
        ],
        out_specs=pl.BlockSpec((mb, do), lambda i: (i, 0)),
        out_shape=jax.ShapeDtypeStruct((n, do), jnp.float32),
    )(s_acc, t_acc, W[:dfh], W[dfh:df], we128)
    return out
